# baseline (speedup 1.0000x reference)
import functools

import jax
import jax.numpy as jnp
from jax import lax
from jax.experimental import pallas as pl
from jax.experimental.pallas import tpu as pltpu
from jax.experimental.pallas import tpu_sc as plsc

D = 1024
H = 1024
N = 8
KS = 1
M = 2048
BM = 256
NW = 32
CH = M // NW
NB = M // BM
T = 24


def _router_body(x_ref, wg_ref, b_ref, pos_ref, w_ref, offs_ref, cnts_ref,
                 se_ref, sb_ref, sv_ref):
    x = x_ref[...]
    wg = wg_ref[...]
    logits = jnp.dot(x, wg, preferred_element_type=jnp.float32)
    s = 1.0 / (1.0 + jnp.exp(-logits))
    biased = s + b_ref[...]
    maxv = jnp.max(biased, axis=1, keepdims=True)
    iota_n = lax.broadcasted_iota(jnp.int32, (M, N), 1)
    e_idx = jnp.min(jnp.where(biased == maxv, iota_n, N), axis=1,
                    keepdims=True)
    onehot = (iota_n == e_idx).astype(jnp.float32)
    s_e = jnp.sum(s * onehot, axis=1, keepdims=True)
    w_ref[...] = jnp.broadcast_to(s_e / (s_e + 1e-9), (M, 128))
    row_i = lax.broadcasted_iota(jnp.int32, (M, M), 0)
    col_i = lax.broadcasted_iota(jnp.int32, (M, M), 1)
    tri = (col_i < row_i).astype(jnp.float32)
    before = jnp.dot(tri, onehot, preferred_element_type=jnp.float32)
    counts = jnp.sum(onehot, axis=0, keepdims=True)
    e_r = lax.broadcasted_iota(jnp.int32, (N, N), 0)
    e_c = lax.broadcasted_iota(jnp.int32, (N, N), 1)
    upper = (e_r < e_c).astype(jnp.float32)
    offs = jnp.dot(counts, upper, preferred_element_type=jnp.float32,
                   precision=lax.Precision.HIGHEST)
    pos = jnp.sum((before + offs) * onehot, axis=1, keepdims=True)
    pos_ref[...] = pos.astype(jnp.int32)
    offs_ref[...] = offs.astype(jnp.int32)
    cnts_ref[...] = counts.astype(jnp.int32)

    F = N * NB
    k_e = lax.broadcasted_iota(jnp.int32, (F, 1), 0) // NB
    k_b = lax.broadcasted_iota(jnp.int32, (F, 1), 0) % NB
    sel = (lax.broadcasted_iota(jnp.int32, (F, N), 1) == k_e)
    selF = sel.astype(jnp.float32)
    start_k = jnp.sum(offs * selF, axis=1, keepdims=True)
    cnt_k = jnp.sum(counts * selF, axis=1, keepdims=True)
    row0_k = (k_b * BM).astype(jnp.float32)
    is_shared_k = k_e >= N - KS
    act = (is_shared_k |
           ((cnt_k > 0) & (start_k < row0_k + BM) &
            (start_k + cnt_k > row0_k))).astype(jnp.float32)
    tri_f = (lax.broadcasted_iota(jnp.int32, (F, F), 1) <
             lax.broadcasted_iota(jnp.int32, (F, F), 0)).astype(jnp.float32)
    rank_k = jnp.dot(tri_f, act, preferred_element_type=jnp.float32)
    slot = lax.broadcasted_iota(jnp.int32, (F, T), 1)
    P = act * (rank_k == slot.astype(jnp.float32)).astype(jnp.float32)
    valid = jnp.sum(P, axis=0, keepdims=True)
    se = jnp.sum(k_e.astype(jnp.float32) * P, axis=0, keepdims=True)
    sb = jnp.sum(k_b.astype(jnp.float32) * P, axis=0, keepdims=True)
    se = jnp.where(valid > 0, se, float(N - 1))
    sb = jnp.where(valid > 0, sb, float(NB - 1))
    se_ref[...] = se.astype(jnp.int32)
    sb_ref[...] = sb.astype(jnp.int32)
    sv_ref[...] = valid.astype(jnp.int32)


def _router_call(x_MD, Wg_DN, b_1N):
    return pl.pallas_call(
        _router_body,
        out_shape=(
            jax.ShapeDtypeStruct((M, 1), jnp.int32),
            jax.ShapeDtypeStruct((M, 128), jnp.float32),
            jax.ShapeDtypeStruct((1, N), jnp.int32),
            jax.ShapeDtypeStruct((1, N), jnp.int32),
            jax.ShapeDtypeStruct((1, T), jnp.int32),
            jax.ShapeDtypeStruct((1, T), jnp.int32),
            jax.ShapeDtypeStruct((1, T), jnp.int32),
        ),
    )(x_MD, Wg_DN, b_1N)


def _dispatch_body(x_hbm, pos_hbm, w_hbm, xs_hbm, ws_hbm,
                   idx_v, rows_v, w_v, sem1, sem2):
    wid = lax.axis_index("s") * 2 + lax.axis_index("c")
    base = wid * CH
    pltpu.sync_copy(pos_hbm.at[pl.ds(base, CH)], idx_v)
    pltpu.sync_copy(x_hbm.at[pl.ds(base, CH)], rows_v)
    pltpu.sync_copy(w_hbm.at[pl.ds(base, CH)], w_v)
    cp1 = pltpu.async_copy(rows_v, xs_hbm.at[idx_v], sem1)
    cp2 = pltpu.async_copy(w_v, ws_hbm.at[idx_v], sem2)
    cp1.wait()
    cp2.wait()


def _dispatch_call(x_MD, pos_M, w_M16):
    mesh = plsc.VectorSubcoreMesh(core_axis_name="c", subcore_axis_name="s")
    f = functools.partial(
        pl.kernel, _dispatch_body, mesh=mesh,
        out_type=(
            jax.ShapeDtypeStruct((M, D), jnp.float32),
            jax.ShapeDtypeStruct((M, 128), jnp.float32),
        ),
        scratch_types=[
            pltpu.VMEM((CH,), jnp.int32),
            pltpu.VMEM((CH, D), jnp.float32),
            pltpu.VMEM((CH, 128), jnp.float32),
            pltpu.SemaphoreType.DMA,
            pltpu.SemaphoreType.DMA,
        ],
    )()
    return f(x_MD, pos_M, w_M16)


def _mlp_body(offs_ref, cnts_ref, se_ref, sb_ref, sv_ref,
              xs_ref, ws_ref, w1_ref, w2_ref, out_ref):
    t = pl.program_id(0)
    e = se_ref[t]
    b = sb_ref[t]
    row0 = b * BM

    @pl.when(t == 0)
    def _init():
        out_ref[...] = jnp.zeros_like(out_ref)

    start = offs_ref[e]
    end = start + cnts_ref[e]
    shared = e >= N - KS
    x = xs_ref[...]
    h = jnp.dot(x, w1_ref[0], preferred_element_type=jnp.float32)
    h1 = h[:, :H]
    h2 = h[:, H:]
    g = h1 * (1.0 / (1.0 + jnp.exp(-h1))) * h2
    o = jnp.dot(g, w2_ref[0], preferred_element_type=jnp.float32)
    ridx = row0 + lax.broadcasted_iota(jnp.int32, (BM, 1), 0)
    inrange = ((ridx >= start) & (ridx < end)).astype(jnp.float32)
    wcol = ws_ref[:, 0:1]
    scale = jnp.where(shared, jnp.ones_like(wcol), wcol * inrange)
    scale = scale * sv_ref[t].astype(jnp.float32)
    out_ref[pl.ds(row0, BM), :] += o * scale


def _mlp_call(xs_MD, ws_M16, W1, W2, offs, cnts, se, sb, sv):
    grid_spec = pltpu.PrefetchScalarGridSpec(
        num_scalar_prefetch=5,
        grid=(T,),
        in_specs=[
            pl.BlockSpec((BM, D), lambda t, o_, c_, se_, sb_, sv_: (sb_[t], 0)),
            pl.BlockSpec((BM, 128), lambda t, o_, c_, se_, sb_, sv_: (sb_[t], 0)),
            pl.BlockSpec((1, D, 2 * H),
                         lambda t, o_, c_, se_, sb_, sv_: (se_[t], 0, 0)),
            pl.BlockSpec((1, H, D),
                         lambda t, o_, c_, se_, sb_, sv_: (se_[t], 0, 0)),
        ],
        out_specs=pl.BlockSpec((M, D), lambda t, o_, c_, se_, sb_, sv_: (0, 0)),
    )
    return pl.pallas_call(
        _mlp_body,
        grid_spec=grid_spec,
        out_shape=jax.ShapeDtypeStruct((M, D), jnp.float32),
        compiler_params=pltpu.CompilerParams(
            dimension_semantics=("arbitrary",),
        ),
    )(offs, cnts, se, sb, sv, xs_MD, ws_M16, W1, W2)


def _combine_body(out_hbm, pos_hbm, y_hbm, idx_v, rows_v, sem):
    wid = lax.axis_index("s") * 2 + lax.axis_index("c")
    base = wid * CH
    pltpu.sync_copy(pos_hbm.at[pl.ds(base, CH)], idx_v)
    pltpu.async_copy(out_hbm.at[idx_v], rows_v, sem).wait()
    pltpu.sync_copy(rows_v, y_hbm.at[pl.ds(base, CH)])


def _combine_call(out_MD, pos_M):
    mesh = plsc.VectorSubcoreMesh(core_axis_name="c", subcore_axis_name="s")
    f = functools.partial(
        pl.kernel, _combine_body, mesh=mesh,
        out_type=jax.ShapeDtypeStruct((M, D), jnp.float32),
        scratch_types=[
            pltpu.VMEM((CH,), jnp.int32),
            pltpu.VMEM((CH, D), jnp.float32),
            pltpu.SemaphoreType.DMA,
        ],
    )()
    return f(out_MD, pos_M)


def kernel(x_BSD, Wg_DN, Wl1_ND2H, Wl2_NHD, biases_N):
    Bq, Sq, Dq = x_BSD.shape
    x_MD = x_BSD.reshape(Bq * Sq, Dq)
    pos2d, w_M16, offs, cnts, se, sb, sv = _router_call(
        x_MD, Wg_DN, biases_N.reshape(1, N))
    pos_M = pos2d.reshape(Bq * Sq)
    xs_MD, ws_M16 = _dispatch_call(x_MD, pos_M, w_M16)
    out_MD = _mlp_call(xs_MD, ws_M16, Wl1_ND2H, Wl2_NHD,
                       offs.reshape(N), cnts.reshape(N),
                       se.reshape(T), sb.reshape(T), sv.reshape(T))
    y_MD = _combine_call(out_MD, pos_M)
    return y_MD.reshape(Bq, Sq, Dq)


# speedup vs baseline: 1.0354x; 1.0354x over previous
import functools

import jax
import jax.numpy as jnp
from jax import lax
from jax.experimental import pallas as pl
from jax.experimental.pallas import tpu as pltpu
from jax.experimental.pallas import tpu_sc as plsc

D = 1024
H = 1024
N = 8
KS = 1
M = 2048
BM = 512
NW = 32
CH = M // NW
NB = M // BM
T = 16


def _router_body(x_ref, wg_ref, b_ref, pos_ref, w_ref, offs_ref, cnts_ref,
                 se_ref, sb_ref, sv_ref):
    x = x_ref[...]
    wg = wg_ref[...]
    logits = jnp.dot(x, wg, preferred_element_type=jnp.float32)
    s = 1.0 / (1.0 + jnp.exp(-logits))
    biased = s + b_ref[...]
    maxv = jnp.max(biased, axis=1, keepdims=True)
    iota_n = lax.broadcasted_iota(jnp.int32, (M, N), 1)
    e_idx = jnp.min(jnp.where(biased == maxv, iota_n, N), axis=1,
                    keepdims=True)
    onehot = (iota_n == e_idx).astype(jnp.float32)
    s_e = jnp.sum(s * onehot, axis=1, keepdims=True)
    w_ref[...] = jnp.broadcast_to(s_e / (s_e + 1e-9), (M, 128))
    row_i = lax.broadcasted_iota(jnp.int32, (M, M), 0)
    col_i = lax.broadcasted_iota(jnp.int32, (M, M), 1)
    tri = (col_i < row_i).astype(jnp.float32)
    before = jnp.dot(tri, onehot, preferred_element_type=jnp.float32)
    counts = jnp.sum(onehot, axis=0, keepdims=True)
    e_r = lax.broadcasted_iota(jnp.int32, (N, N), 0)
    e_c = lax.broadcasted_iota(jnp.int32, (N, N), 1)
    upper = (e_r < e_c).astype(jnp.float32)
    offs = jnp.dot(counts, upper, preferred_element_type=jnp.float32,
                   precision=lax.Precision.HIGHEST)
    pos = jnp.sum((before + offs) * onehot, axis=1, keepdims=True)
    pos_ref[...] = pos.astype(jnp.int32)
    offs_ref[...] = offs.astype(jnp.int32)
    cnts_ref[...] = counts.astype(jnp.int32)

    F = N * NB
    k_e = lax.broadcasted_iota(jnp.int32, (F, 1), 0) // NB
    k_b = lax.broadcasted_iota(jnp.int32, (F, 1), 0) % NB
    sel = (lax.broadcasted_iota(jnp.int32, (F, N), 1) == k_e)
    selF = sel.astype(jnp.float32)
    start_k = jnp.sum(offs * selF, axis=1, keepdims=True)
    cnt_k = jnp.sum(counts * selF, axis=1, keepdims=True)
    row0_k = (k_b * BM).astype(jnp.float32)
    is_shared_k = k_e >= N - KS
    act = (is_shared_k |
           ((cnt_k > 0) & (start_k < row0_k + BM) &
            (start_k + cnt_k > row0_k))).astype(jnp.float32)
    tri_f = (lax.broadcasted_iota(jnp.int32, (F, F), 1) <
             lax.broadcasted_iota(jnp.int32, (F, F), 0)).astype(jnp.float32)
    rank_k = jnp.dot(tri_f, act, preferred_element_type=jnp.float32)
    slot = lax.broadcasted_iota(jnp.int32, (F, T), 1)
    P = act * (rank_k == slot.astype(jnp.float32)).astype(jnp.float32)
    valid = jnp.sum(P, axis=0, keepdims=True)
    se = jnp.sum(k_e.astype(jnp.float32) * P, axis=0, keepdims=True)
    sb = jnp.sum(k_b.astype(jnp.float32) * P, axis=0, keepdims=True)
    se = jnp.where(valid > 0, se, float(N - 1))
    sb = jnp.where(valid > 0, sb, float(NB - 1))
    se_ref[...] = se.astype(jnp.int32)
    sb_ref[...] = sb.astype(jnp.int32)
    sv_ref[...] = valid.astype(jnp.int32)


def _router_call(x_MD, Wg_DN, b_1N):
    return pl.pallas_call(
        _router_body,
        out_shape=(
            jax.ShapeDtypeStruct((M, 1), jnp.int32),
            jax.ShapeDtypeStruct((M, 128), jnp.float32),
            jax.ShapeDtypeStruct((1, N), jnp.int32),
            jax.ShapeDtypeStruct((1, N), jnp.int32),
            jax.ShapeDtypeStruct((1, T), jnp.int32),
            jax.ShapeDtypeStruct((1, T), jnp.int32),
            jax.ShapeDtypeStruct((1, T), jnp.int32),
        ),
    )(x_MD, Wg_DN, b_1N)


def _dispatch_body(x_hbm, pos_hbm, w_hbm, xs_hbm, ws_hbm,
                   idx_v, rows_v, w_v, sem1, sem2):
    wid = lax.axis_index("s") * 2 + lax.axis_index("c")
    base = wid * CH
    pltpu.sync_copy(pos_hbm.at[pl.ds(base, CH)], idx_v)
    pltpu.sync_copy(x_hbm.at[pl.ds(base, CH)], rows_v)
    pltpu.sync_copy(w_hbm.at[pl.ds(base, CH)], w_v)
    cp1 = pltpu.async_copy(rows_v, xs_hbm.at[idx_v], sem1)
    cp2 = pltpu.async_copy(w_v, ws_hbm.at[idx_v], sem2)
    cp1.wait()
    cp2.wait()


def _dispatch_call(x_MD, pos_M, w_M16):
    mesh = plsc.VectorSubcoreMesh(core_axis_name="c", subcore_axis_name="s")
    f = functools.partial(
        pl.kernel, _dispatch_body, mesh=mesh,
        out_type=(
            jax.ShapeDtypeStruct((M, D), jnp.float32),
            jax.ShapeDtypeStruct((M, 128), jnp.float32),
        ),
        scratch_types=[
            pltpu.VMEM((CH,), jnp.int32),
            pltpu.VMEM((CH, D), jnp.float32),
            pltpu.VMEM((CH, 128), jnp.float32),
            pltpu.SemaphoreType.DMA,
            pltpu.SemaphoreType.DMA,
        ],
    )()
    return f(x_MD, pos_M, w_M16)


def _mlp_body(offs_ref, cnts_ref, se_ref, sb_ref, sv_ref,
              xs_ref, ws_ref, w1_ref, w2_ref, out_ref):
    t = pl.program_id(0)
    e = se_ref[t]
    b = sb_ref[t]
    row0 = b * BM

    @pl.when(t == 0)
    def _init():
        out_ref[...] = jnp.zeros_like(out_ref)

    start = offs_ref[e]
    end = start + cnts_ref[e]
    shared = e >= N - KS
    x = xs_ref[...]
    h = jnp.dot(x, w1_ref[0], preferred_element_type=jnp.float32)
    h1 = h[:, :H]
    h2 = h[:, H:]
    g = h1 * (1.0 / (1.0 + jnp.exp(-h1))) * h2
    o = jnp.dot(g, w2_ref[0], preferred_element_type=jnp.float32)
    ridx = row0 + lax.broadcasted_iota(jnp.int32, (BM, 1), 0)
    inrange = ((ridx >= start) & (ridx < end)).astype(jnp.float32)
    wcol = ws_ref[:, 0:1]
    scale = jnp.where(shared, jnp.ones_like(wcol), wcol * inrange)
    scale = scale * sv_ref[t].astype(jnp.float32)
    out_ref[pl.ds(row0, BM), :] += o * scale


def _mlp_call(xs_MD, ws_M16, W1, W2, offs, cnts, se, sb, sv):
    grid_spec = pltpu.PrefetchScalarGridSpec(
        num_scalar_prefetch=5,
        grid=(T,),
        in_specs=[
            pl.BlockSpec((BM, D), lambda t, o_, c_, se_, sb_, sv_: (sb_[t], 0)),
            pl.BlockSpec((BM, 128), lambda t, o_, c_, se_, sb_, sv_: (sb_[t], 0)),
            pl.BlockSpec((1, D, 2 * H),
                         lambda t, o_, c_, se_, sb_, sv_: (se_[t], 0, 0)),
            pl.BlockSpec((1, H, D),
                         lambda t, o_, c_, se_, sb_, sv_: (se_[t], 0, 0)),
        ],
        out_specs=pl.BlockSpec((M, D), lambda t, o_, c_, se_, sb_, sv_: (0, 0)),
    )
    return pl.pallas_call(
        _mlp_body,
        grid_spec=grid_spec,
        out_shape=jax.ShapeDtypeStruct((M, D), jnp.float32),
        compiler_params=pltpu.CompilerParams(
            dimension_semantics=("arbitrary",),
        ),
    )(offs, cnts, se, sb, sv, xs_MD, ws_M16, W1, W2)


def _combine_body(out_hbm, pos_hbm, y_hbm, idx_v, rows_v, sem):
    wid = lax.axis_index("s") * 2 + lax.axis_index("c")
    base = wid * CH
    pltpu.sync_copy(pos_hbm.at[pl.ds(base, CH)], idx_v)
    pltpu.async_copy(out_hbm.at[idx_v], rows_v, sem).wait()
    pltpu.sync_copy(rows_v, y_hbm.at[pl.ds(base, CH)])


def _combine_call(out_MD, pos_M):
    mesh = plsc.VectorSubcoreMesh(core_axis_name="c", subcore_axis_name="s")
    f = functools.partial(
        pl.kernel, _combine_body, mesh=mesh,
        out_type=jax.ShapeDtypeStruct((M, D), jnp.float32),
        scratch_types=[
            pltpu.VMEM((CH,), jnp.int32),
            pltpu.VMEM((CH, D), jnp.float32),
            pltpu.SemaphoreType.DMA,
        ],
    )()
    return f(out_MD, pos_M)


def kernel(x_BSD, Wg_DN, Wl1_ND2H, Wl2_NHD, biases_N):
    Bq, Sq, Dq = x_BSD.shape
    x_MD = x_BSD.reshape(Bq * Sq, Dq)
    pos2d, w_M16, offs, cnts, se, sb, sv = _router_call(
        x_MD, Wg_DN, biases_N.reshape(1, N))
    pos_M = pos2d.reshape(Bq * Sq)
    xs_MD, ws_M16 = _dispatch_call(x_MD, pos_M, w_M16)
    out_MD = _mlp_call(xs_MD, ws_M16, Wl1_ND2H, Wl2_NHD,
                       offs.reshape(N), cnts.reshape(N),
                       se.reshape(T), sb.reshape(T), sv.reshape(T))
    y_MD = _combine_call(out_MD, pos_M)
    return y_MD.reshape(Bq, Sq, Dq)
